# Initial kernel scaffold; baseline (speedup 1.0000x reference)
import jax
import jax.numpy as jnp
from jax.experimental import pallas as pl


def kernel(x, edge_index, batch, Wc1, bc1, W11, b11, W12, b12, W13, b13, Wc2, bc2, W21, b21, W22, b22, W23, b23, Wfc2, bfc2):
    raise NotImplementedError("write your pallas kernel here")


# speedup vs baseline: 10.1350x; 10.1350x over previous
import functools

import jax
import jax.numpy as jnp
from jax import lax
from jax.experimental import pallas as pl
from jax.experimental.pallas import tpu as pltpu
from jax.experimental.pallas import tpu_sc as plsc

_NC = 2
_NS = 16
_NW = _NC * _NS
_G = 64


def _make_segsum(n, e, d):
    epw = e // _NW
    c = 125
    nchunk = epw // c
    r0 = (n // _NS) // 8 * 8
    rlast = n - (_NS - 1) * r0
    mesh = plsc.VectorSubcoreMesh(core_axis_name="c", subcore_axis_name="s")

    @functools.partial(
        pl.kernel,
        out_type=jax.ShapeDtypeStruct((_NC, n, d), jnp.float32),
        mesh=mesh,
        compiler_params=pltpu.CompilerParams(use_tc_tiling_on_sc=False),
        scratch_types=[
            pltpu.VMEM((nchunk, c), jnp.int32),
            pltpu.VMEM((nchunk, c), jnp.int32),
            pltpu.VMEM((c, d), jnp.float32),
            pltpu.VMEM_SHARED((n, d), jnp.float32),
            pltpu.SemaphoreType.DMA,
            pltpu.SemaphoreType.DMA,
        ],
    )
    def segsum(src_hbm, dst_hbm, y_hbm, zeros_hbm, out_hbm,
               idx_s, idx_d, rows, agg_sh, sem_g, sem_i):
        cid = lax.axis_index("c")
        sid = lax.axis_index("s")
        wid = sid * _NC + cid
        @pl.when(sid < _NS - 1)
        def _():
            pltpu.sync_copy(zeros_hbm.at[pl.ds(0, r0)],
                            agg_sh.at[pl.ds(sid * r0, r0)])

        @pl.when(sid == _NS - 1)
        def _():
            pltpu.sync_copy(zeros_hbm,
                            agg_sh.at[pl.ds((_NS - 1) * r0, rlast)])
        pltpu.async_copy(src_hbm.at[wid], idx_s, sem_i)
        pltpu.async_copy(dst_hbm.at[wid], idx_d, sem_i).wait()
        pltpu.make_async_copy(src_hbm.at[wid], idx_s, sem_i).wait()
        plsc.subcore_barrier()

        def body(ch, carry):
            pltpu.async_copy(y_hbm.at[idx_s.at[ch]], rows, sem_g).wait()
            pltpu.sync_copy(rows, agg_sh.at[idx_d.at[ch]], add=True)
            return carry

        lax.fori_loop(0, nchunk, body, 0, unroll=False)
        plsc.subcore_barrier()

        @pl.when(sid < _NS - 1)
        def _():
            pltpu.sync_copy(agg_sh.at[pl.ds(sid * r0, r0)],
                            out_hbm.at[cid, pl.ds(sid * r0, r0)])

        @pl.when(sid == _NS - 1)
        def _():
            pltpu.sync_copy(agg_sh.at[pl.ds((_NS - 1) * r0, rlast)],
                            out_hbm.at[cid, pl.ds((_NS - 1) * r0, rlast)])

    return segsum


def _tc1_body(x_ref, w_ref, b_ref, y_ref, am_ref):
    z = jnp.dot(x_ref[...], w_ref[...], preferred_element_type=jnp.float32)
    z = z + b_ref[...]
    y_ref[...] = z[:, :64]
    a = jnp.maximum(z[:, 64:128], 0.0)
    m = jnp.maximum(z[:, 128:144], 0.0) * jnp.maximum(z[:, 144:160], 0.0)
    am_ref[...] = jnp.concatenate([a, m], axis=1)


def _tc2_body(am_ref, agg_ref, bc_ref, wa_ref, wc_ref, wm_ref, b_ref,
              y_ref, am2_ref):
    c = jnp.maximum(agg_ref[0] + agg_ref[1] + bc_ref[...], 0.0)
    a = am_ref[:, :64]
    m = am_ref[:, 64:80]
    z = (jnp.dot(a, wa_ref[...], preferred_element_type=jnp.float32)
         + jnp.dot(c, wc_ref[...], preferred_element_type=jnp.float32)
         + jnp.dot(m, wm_ref[...], preferred_element_type=jnp.float32)
         + b_ref[...])
    y_ref[...] = z[:, :64]
    a2 = jnp.maximum(z[:, 64:128], 0.0)
    m2 = jnp.maximum(z[:, 128:144], 0.0) * jnp.maximum(z[:, 144:160], 0.0)
    am2_ref[...] = jnp.concatenate([a2, m2], axis=1)


def _tc3_body(nb, am_ref, agg_ref, bc_ref, batch_ref, wfc_ref, bfc_ref,
              out_ref, sum_ref, mx_ref, cnt_ref):
    i = pl.program_id(0)

    @pl.when(i == 0)
    def _():
        sum_ref[...] = jnp.zeros_like(sum_ref)
        mx_ref[...] = jnp.zeros_like(mx_ref)
        cnt_ref[...] = jnp.zeros_like(cnt_ref)

    c = jnp.maximum(agg_ref[0] + agg_ref[1] + bc_ref[...], 0.0)
    h2 = jnp.concatenate([am_ref[:, :64], c, am_ref[:, 64:80]], axis=1)
    bsz = h2.shape[0]
    bid = batch_ref[0, 0, :].reshape(bsz, 1)
    g_lo = batch_ref[0, 0, 0]
    g_hi = batch_ref[0, 0, bsz - 1]

    def seg(g, carry):
        msk = (bid == g).astype(jnp.float32)
        mh = h2 * msk
        s = jnp.sum(mh, axis=0, keepdims=True)
        mx = jnp.max(mh, axis=0, keepdims=True)
        cnt = jnp.sum(msk)
        sum_ref[pl.ds(g, 1), :] += s
        mx_ref[pl.ds(g, 1), :] = jnp.maximum(mx_ref[pl.ds(g, 1), :], mx)
        cnt_ref[pl.ds(g, 1), :] += cnt
        return carry

    lax.fori_loop(g_lo, g_hi + 1, seg, 0)

    @pl.when(i == nb - 1)
    def _():
        mean = sum_ref[...] / jnp.maximum(cnt_ref[...], 1.0)
        pooled = jnp.concatenate([mean, mx_ref[...]], axis=1)
        logits = jnp.dot(pooled, wfc_ref[...],
                         preferred_element_type=jnp.float32) + bfc_ref[...]
        m = jnp.max(logits, axis=1, keepdims=True)
        lse = m + jnp.log(jnp.sum(jnp.exp(logits - m), axis=1, keepdims=True))
        out_ref[...] = logits - lse


def kernel(x, edge_index, batch, Wc1, bc1, W11, b11, W12, b12, W13, b13,
           Wc2, bc2, W21, b21, W22, b22, W23, b23, Wfc2, bfc2):
    n, dx = x.shape
    e = edge_index.shape[1]
    epw = e // _NW
    c = 125
    nchunk = epw // c

    src3 = edge_index[0].reshape(_NW, nchunk, c)
    dst3 = edge_index[1].reshape(_NW, nchunk, c)
    zeros64 = jnp.zeros((n - (_NS - 1) * ((n // _NS) // 8 * 8), 64),
                        jnp.float32)

    W1 = jnp.concatenate([Wc1, W11, W12, W13], axis=1)
    bz1 = jnp.concatenate(
        [jnp.zeros_like(bc1), b11, b12, b13])[None, :]
    W2 = jnp.concatenate([Wc2, W21, W22, W23], axis=1)
    W2a, W2c, W2m = W2[:64], W2[64:128], W2[128:144]
    bz2 = jnp.concatenate(
        [jnp.zeros((64,), jnp.float32), b21, b22, b23])[None, :]

    bsz = 1000
    nb = n // bsz
    full = lambda shape: pl.BlockSpec(shape, lambda i: tuple(0 for _ in shape))

    y1, am1 = pl.pallas_call(
        _tc1_body,
        grid=(nb,),
        in_specs=[
            pl.BlockSpec((bsz, dx), lambda i: (i, 0)),
            full((dx, 160)),
            full((1, 160)),
        ],
        out_specs=[
            pl.BlockSpec((bsz, 64), lambda i: (i, 0)),
            pl.BlockSpec((bsz, 80), lambda i: (i, 0)),
        ],
        out_shape=[
            jax.ShapeDtypeStruct((n, 64), jnp.float32),
            jax.ShapeDtypeStruct((n, 80), jnp.float32),
        ],
    )(x, W1, bz1)

    agg1 = _make_segsum(n, e, 64)(src3, dst3, y1, zeros64)

    y2, am2 = pl.pallas_call(
        _tc2_body,
        grid=(nb,),
        in_specs=[
            pl.BlockSpec((bsz, 80), lambda i: (i, 0)),
            pl.BlockSpec((_NC, bsz, 64), lambda i: (0, i, 0)),
            full((1, 64)),
            full((64, 160)),
            full((64, 160)),
            full((16, 160)),
            full((1, 160)),
        ],
        out_specs=[
            pl.BlockSpec((bsz, 64), lambda i: (i, 0)),
            pl.BlockSpec((bsz, 80), lambda i: (i, 0)),
        ],
        out_shape=[
            jax.ShapeDtypeStruct((n, 64), jnp.float32),
            jax.ShapeDtypeStruct((n, 80), jnp.float32),
        ],
    )(am1, agg1, bc1[None, :], W2a, W2c, W2m, bz2)

    agg2 = _make_segsum(n, e, 64)(src3, dst3, y2, zeros64)

    batch3 = batch.reshape(nb, 1, bsz)
    out = pl.pallas_call(
        functools.partial(_tc3_body, nb),
        grid=(nb,),
        in_specs=[
            pl.BlockSpec((bsz, 80), lambda i: (i, 0)),
            pl.BlockSpec((_NC, bsz, 64), lambda i: (0, i, 0)),
            full((1, 64)),
            pl.BlockSpec((1, 1, bsz), lambda i: (i, 0, 0)),
            full((288, 2)),
            full((1, 2)),
        ],
        out_specs=pl.BlockSpec((_G, 2), lambda i: (0, 0)),
        out_shape=jax.ShapeDtypeStruct((_G, 2), jnp.float32),
        scratch_shapes=[
            pltpu.VMEM((_G, 144), jnp.float32),
            pltpu.VMEM((_G, 144), jnp.float32),
            pltpu.VMEM((_G, 144), jnp.float32),
        ],
    )(am2, agg2, bc2[None, :], batch3, Wfc2, bfc2[None, :])
    return out
